# baseline (speedup 1.0000x reference)
import functools

import jax
import jax.numpy as jnp
from jax import lax
from jax.experimental import pallas as pl
from jax.experimental.pallas import tpu as pltpu
from jax.experimental.pallas import tpu_sc as plsc

LEVELS = 16
FEATURE_DIM = 256
NUM_WORDS = 10000
DICT_SIZE = 256
BATCH = 8192


_W_BLK = 1000


_CODE_W = 128


def _argmax_body(feats_ref, code_hbm, code_v, sem):
    cols = []
    for l in range(LEVELS):
        x = feats_ref[l]
        m = jnp.max(x, axis=-1, keepdims=True)
        lane_f = lax.broadcasted_iota(jnp.int32, x.shape, 1).astype(jnp.float32)
        cand = jnp.where(x == m, lane_f, float(DICT_SIZE))
        code_f = jnp.min(cand, axis=-1, keepdims=True)
        cols.append(code_f.astype(jnp.int32))
    cols.append(jnp.zeros((x.shape[0], _CODE_W - LEVELS), jnp.int32))
    code_v[...] = jnp.concatenate(cols, axis=1)
    i = pl.program_id(0)
    cp = pltpu.make_async_copy(
        code_v, code_hbm.at[pl.ds(i * _W_BLK, _W_BLK), :], sem
    )
    cp.start()
    cp.wait()


def _compute_codes(feats):
    return pl.pallas_call(
        _argmax_body,
        grid=(NUM_WORDS // _W_BLK,),
        in_specs=[
            pl.BlockSpec((LEVELS, _W_BLK, DICT_SIZE), lambda i: (0, i, 0)),
        ],
        out_specs=pl.BlockSpec(memory_space=pl.ANY),
        out_shape=jax.ShapeDtypeStruct((NUM_WORDS, _CODE_W), jnp.int32),
        scratch_shapes=[
            pltpu.VMEM((_W_BLK, _CODE_W), jnp.int32),
            pltpu.SemaphoreType.DMA,
        ],
    )(feats)


def _gather_codes(code_all, idx):
    info = plsc.get_sparse_core_info()
    nc, ns = info.num_cores, info.num_subcores
    nw = nc * ns
    b_per_w = BATCH // nw
    n_chunk = b_per_w // 128

    mesh = plsc.VectorSubcoreMesh(core_axis_name="c", subcore_axis_name="s")

    @functools.partial(
        pl.kernel,
        mesh=mesh,
        out_type=jax.ShapeDtypeStruct((BATCH, _CODE_W), jnp.int32),
        scratch_types=[
            pltpu.VMEM((n_chunk, 128), jnp.int32),
            pltpu.VMEM((b_per_w, _CODE_W), jnp.int32),
            pltpu.SemaphoreType.DMA,
        ],
    )
    def k(code_hbm, idx_hbm, out_hbm, idx_v, rows_v, sem):
        wid = lax.axis_index("s") * nc + lax.axis_index("c")
        base = wid * b_per_w
        for j in range(n_chunk):
            pltpu.sync_copy(idx_hbm.at[pl.ds(base + j * 128, 128)], idx_v.at[j])
        copies = [
            pltpu.async_copy(
                code_hbm.at[idx_v.at[j]], rows_v.at[pl.ds(j * 128, 128)], sem
            )
            for j in range(n_chunk)
        ]
        for c in copies:
            c.wait()
        pltpu.sync_copy(rows_v, out_hbm.at[pl.ds(base, b_per_w)])

    return k(code_all, idx)


_B_BLK = 512


def _expand_body(code_hbm, dict_hbm, out_ref, code_v, dict_v, csem, dsem):
    i = pl.program_id(0)

    @pl.when(i == 0)
    def _():
        pltpu.make_async_copy(dict_hbm, dict_v, dsem).start()

    cp = pltpu.make_async_copy(
        code_hbm.at[pl.ds(i * _B_BLK, _B_BLK), :], code_v, csem
    )
    cp.start()

    @pl.when(i == 0)
    def _():
        pltpu.make_async_copy(dict_hbm, dict_v, dsem).wait()

    cp.wait()

    lane = lax.broadcasted_iota(jnp.int32, (_B_BLK, DICT_SIZE), 1)
    for l in range(LEVELS):
        c = code_v[:, l : l + 1]
        oh = (c == lane).astype(jnp.float32)
        out_ref[:, l * FEATURE_DIM : (l + 1) * FEATURE_DIM] = jnp.dot(
            oh, dict_v[l], preferred_element_type=jnp.float32
        )


def _expand(code_sel, dictionary):
    return pl.pallas_call(
        _expand_body,
        grid=(BATCH // _B_BLK,),
        in_specs=[
            pl.BlockSpec(memory_space=pl.ANY),
            pl.BlockSpec(memory_space=pl.ANY),
        ],
        out_specs=pl.BlockSpec((_B_BLK, LEVELS * FEATURE_DIM), lambda i: (i, 0)),
        out_shape=jax.ShapeDtypeStruct((BATCH, LEVELS * FEATURE_DIM), jnp.float32),
        scratch_shapes=[
            pltpu.VMEM((_B_BLK, _CODE_W), jnp.int32),
            pltpu.VMEM((LEVELS, DICT_SIZE, FEATURE_DIM), jnp.float32),
            pltpu.SemaphoreType.DMA,
            pltpu.SemaphoreType.DMA,
        ],
    )(code_sel, dictionary)


def kernel(idx, dictionary, feats):
    code_all = _compute_codes(feats)
    code_sel = _gather_codes(code_all, idx.astype(jnp.int32))
    return _expand(code_sel, dictionary)


# speedup vs baseline: 1.0647x; 1.0647x over previous
import functools

import jax
import jax.numpy as jnp
from jax import lax
from jax.experimental import pallas as pl
from jax.experimental.pallas import tpu as pltpu
from jax.experimental.pallas import tpu_sc as plsc

LEVELS = 16
FEATURE_DIM = 256
NUM_WORDS = 10000
DICT_SIZE = 256
BATCH = 8192


_W_BLK = 1000


_CODE_W = LEVELS


def _argmax_body(feats_ref, code_ref):
    cols = []
    for l in range(LEVELS):
        x = feats_ref[l]
        m = jnp.max(x, axis=-1, keepdims=True)
        lane_f = lax.broadcasted_iota(jnp.int32, x.shape, 1).astype(jnp.float32)
        cand = jnp.where(x == m, lane_f, float(DICT_SIZE))
        code_f = jnp.min(cand, axis=-1, keepdims=True)
        cols.append(code_f.astype(jnp.int32))
    code_ref[...] = jnp.concatenate(cols, axis=1)


def _compute_codes(feats):
    return pl.pallas_call(
        _argmax_body,
        grid=(NUM_WORDS // _W_BLK,),
        in_specs=[
            pl.BlockSpec((LEVELS, _W_BLK, DICT_SIZE), lambda i: (0, i, 0)),
        ],
        out_specs=pl.BlockSpec((_W_BLK, _CODE_W), lambda i: (i, 0)),
        out_shape=jax.ShapeDtypeStruct((NUM_WORDS, _CODE_W), jnp.int32),
    )(feats)


def _gather_codes(code_all, idx):
    info = plsc.get_sparse_core_info()
    nc, ns = info.num_cores, info.num_subcores
    nw = nc * ns
    b_per_w = BATCH // nw
    n_chunk = b_per_w // 128

    mesh = plsc.VectorSubcoreMesh(core_axis_name="c", subcore_axis_name="s")

    @functools.partial(
        pl.kernel,
        mesh=mesh,
        out_type=jax.ShapeDtypeStruct((BATCH, _CODE_W), jnp.int32),
        scratch_types=[
            pltpu.VMEM((n_chunk, 128), jnp.int32),
            pltpu.VMEM((b_per_w, _CODE_W), jnp.int32),
            pltpu.SemaphoreType.DMA,
        ],
        compiler_params=pltpu.CompilerParams(use_tc_tiling_on_sc=False),
    )
    def k(code_hbm, idx_hbm, out_hbm, idx_v, rows_v, sem):
        wid = lax.axis_index("s") * nc + lax.axis_index("c")
        base = wid * b_per_w
        for j in range(n_chunk):
            pltpu.sync_copy(idx_hbm.at[pl.ds(base + j * 128, 128)], idx_v.at[j])
        copies = [
            pltpu.async_copy(
                code_hbm.at[idx_v.at[j]], rows_v.at[pl.ds(j * 128, 128)], sem
            )
            for j in range(n_chunk)
        ]
        for c in copies:
            c.wait()
        pltpu.sync_copy(rows_v, out_hbm.at[pl.ds(base, b_per_w)])

    return k(code_all, idx)


_B_BLK = 512


def _expand_body(code_ref, dict_ref, out_ref):
    lane = lax.broadcasted_iota(jnp.int32, (_B_BLK, DICT_SIZE), 1)
    for l in range(LEVELS):
        c = code_ref[:, l : l + 1]
        oh = (c == lane).astype(jnp.float32)
        out_ref[:, l * FEATURE_DIM : (l + 1) * FEATURE_DIM] = jnp.dot(
            oh, dict_ref[l], preferred_element_type=jnp.float32
        )


def _expand(code_sel, dictionary):
    return pl.pallas_call(
        _expand_body,
        grid=(BATCH // _B_BLK,),
        in_specs=[
            pl.BlockSpec((_B_BLK, _CODE_W), lambda i: (i, 0)),
            pl.BlockSpec((LEVELS, DICT_SIZE, FEATURE_DIM), lambda i: (0, 0, 0)),
        ],
        out_specs=pl.BlockSpec((_B_BLK, LEVELS * FEATURE_DIM), lambda i: (i, 0)),
        out_shape=jax.ShapeDtypeStruct((BATCH, LEVELS * FEATURE_DIM), jnp.float32),
    )(code_sel, dictionary)


def kernel(idx, dictionary, feats):
    code_all = _compute_codes(feats)
    code_sel = _gather_codes(code_all, idx.astype(jnp.int32))
    return _expand(code_sel, dictionary)


# speedup vs baseline: 1.1133x; 1.0457x over previous
import functools

import jax
import jax.numpy as jnp
from jax import lax
from jax.experimental import pallas as pl
from jax.experimental.pallas import tpu as pltpu
from jax.experimental.pallas import tpu_sc as plsc

LEVELS = 16
FEATURE_DIM = 256
NUM_WORDS = 10000
DICT_SIZE = 256
BATCH = 8192


_W_BLK = 1000


_CODE_W = 128


def _argmax_body(feats_ref, code_ref):
    cols = []
    for l in range(LEVELS):
        x = feats_ref[l]
        m = jnp.max(x, axis=-1, keepdims=True)
        lane_f = lax.broadcasted_iota(jnp.int32, x.shape, 1).astype(jnp.float32)
        cand = jnp.where(x == m, lane_f, float(DICT_SIZE))
        code_f = jnp.min(cand, axis=-1, keepdims=True)
        cols.append(code_f.astype(jnp.int32))
    cols.append(jnp.zeros((x.shape[0], _CODE_W - LEVELS), jnp.int32))
    code_ref[...] = jnp.concatenate(cols, axis=1)


def _compute_codes(feats):
    return pl.pallas_call(
        _argmax_body,
        grid=(NUM_WORDS // _W_BLK,),
        in_specs=[
            pl.BlockSpec((LEVELS, _W_BLK, DICT_SIZE), lambda i: (0, i, 0)),
        ],
        out_specs=pl.BlockSpec((_W_BLK, _CODE_W), lambda i: (i, 0)),
        out_shape=jax.ShapeDtypeStruct((NUM_WORDS, _CODE_W), jnp.int32),
    )(feats)


def _gather_codes(code_all, idx):
    info = plsc.get_sparse_core_info()
    nc, ns = info.num_cores, info.num_subcores
    nw = nc * ns
    b_per_w = BATCH // nw
    n_chunk = b_per_w // 128

    mesh = plsc.VectorSubcoreMesh(core_axis_name="c", subcore_axis_name="s")

    @functools.partial(
        pl.kernel,
        mesh=mesh,
        out_type=jax.ShapeDtypeStruct((BATCH, _CODE_W), jnp.int32),
        scratch_types=[
            pltpu.VMEM((n_chunk, 128), jnp.int32),
            pltpu.VMEM((b_per_w, _CODE_W), jnp.int32),
            pltpu.SemaphoreType.DMA,
        ],
    )
    def k(code_hbm, idx_hbm, out_hbm, idx_v, rows_v, sem):
        wid = lax.axis_index("s") * nc + lax.axis_index("c")
        pltpu.sync_copy(idx_hbm.at[pl.ds(wid * n_chunk, n_chunk)], idx_v)
        copies = [
            pltpu.async_copy(
                code_hbm.at[idx_v.at[j]], rows_v.at[pl.ds(j * 128, 128)], sem
            )
            for j in range(n_chunk)
        ]
        for c in copies:
            c.wait()
        pltpu.sync_copy(rows_v, out_hbm.at[pl.ds(wid * b_per_w, b_per_w)])

    return k(code_all, idx.reshape(BATCH // 128, 128))


_B_BLK = 512


def _expand_body(code_ref, dict_ref, out_ref):
    lane = lax.broadcasted_iota(jnp.int32, (_B_BLK, DICT_SIZE), 1)
    for l in range(LEVELS):
        c = code_ref[:, l : l + 1]
        oh = (c == lane).astype(jnp.float32)
        out_ref[:, l * FEATURE_DIM : (l + 1) * FEATURE_DIM] = jnp.dot(
            oh, dict_ref[l], preferred_element_type=jnp.float32
        )


def _expand(code_sel, dictionary):
    return pl.pallas_call(
        _expand_body,
        grid=(BATCH // _B_BLK,),
        in_specs=[
            pl.BlockSpec((_B_BLK, _CODE_W), lambda i: (i, 0)),
            pl.BlockSpec((LEVELS, DICT_SIZE, FEATURE_DIM), lambda i: (0, 0, 0)),
        ],
        out_specs=pl.BlockSpec((_B_BLK, LEVELS * FEATURE_DIM), lambda i: (i, 0)),
        out_shape=jax.ShapeDtypeStruct((BATCH, LEVELS * FEATURE_DIM), jnp.float32),
    )(code_sel, dictionary)


def kernel(idx, dictionary, feats):
    code_all = _compute_codes(feats)
    code_sel = _gather_codes(code_all, idx.astype(jnp.int32))
    return _expand(code_sel, dictionary)


# speedup vs baseline: 1.1184x; 1.0046x over previous
import functools

import jax
import jax.numpy as jnp
from jax import lax
from jax.experimental import pallas as pl
from jax.experimental.pallas import tpu as pltpu
from jax.experimental.pallas import tpu_sc as plsc

LEVELS = 16
FEATURE_DIM = 256
NUM_WORDS = 10000
DICT_SIZE = 256
BATCH = 8192


_W_BLK = 1000


_CODE_W = 128


_N_WSTEP = NUM_WORDS // _W_BLK


def _argmax_body(feats_ref, code_hbm, code_v, sem):
    i = pl.program_id(0)
    slot = lax.rem(i, 2)

    def _slot_copy(s, step):
        return pltpu.make_async_copy(
            code_v.at[s],
            code_hbm.at[pl.ds(step * _W_BLK, _W_BLK), :],
            sem.at[s],
        )

    @pl.when(i >= 2)
    def _():
        _slot_copy(slot, i - 2).wait()

    cols = []
    for l in range(LEVELS):
        x = feats_ref[l]
        m = jnp.max(x, axis=-1, keepdims=True)
        lane_f = lax.broadcasted_iota(jnp.int32, x.shape, 1).astype(jnp.float32)
        cand = jnp.where(x == m, lane_f, float(DICT_SIZE))
        code_f = jnp.min(cand, axis=-1, keepdims=True)
        cols.append(code_f.astype(jnp.int32))
    cols.append(jnp.zeros((x.shape[0], _CODE_W - LEVELS), jnp.int32))
    code_v[slot] = jnp.concatenate(cols, axis=1)
    _slot_copy(slot, i).start()

    @pl.when(i == _N_WSTEP - 1)
    def _():
        _slot_copy(1 - slot, i - 1).wait()
        _slot_copy(slot, i).wait()


def _compute_codes(feats):
    return pl.pallas_call(
        _argmax_body,
        grid=(_N_WSTEP,),
        in_specs=[
            pl.BlockSpec((LEVELS, _W_BLK, DICT_SIZE), lambda i: (0, i, 0)),
        ],
        out_specs=pl.BlockSpec(memory_space=pl.ANY),
        out_shape=jax.ShapeDtypeStruct((NUM_WORDS, _CODE_W), jnp.int32),
        scratch_shapes=[
            pltpu.VMEM((2, _W_BLK, _CODE_W), jnp.int32),
            pltpu.SemaphoreType.DMA((2,)),
        ],
    )(feats)


def _gather_codes(code_all, idx):
    info = plsc.get_sparse_core_info()
    nc, ns = info.num_cores, info.num_subcores
    nw = nc * ns
    b_per_w = BATCH // nw
    n_chunk = b_per_w // 128

    mesh = plsc.VectorSubcoreMesh(core_axis_name="c", subcore_axis_name="s")

    @functools.partial(
        pl.kernel,
        mesh=mesh,
        out_type=jax.ShapeDtypeStruct((BATCH, _CODE_W), jnp.int32),
        scratch_types=[
            pltpu.VMEM((n_chunk, 128), jnp.int32),
            pltpu.VMEM((b_per_w, _CODE_W), jnp.int32),
            pltpu.SemaphoreType.DMA,
        ],
    )
    def k(code_hbm, idx_hbm, out_hbm, idx_v, rows_v, sem):
        wid = lax.axis_index("s") * nc + lax.axis_index("c")
        pltpu.sync_copy(idx_hbm.at[pl.ds(wid * n_chunk, n_chunk)], idx_v)
        copies = [
            pltpu.async_copy(
                code_hbm.at[idx_v.at[j]], rows_v.at[pl.ds(j * 128, 128)], sem
            )
            for j in range(n_chunk)
        ]
        for c in copies:
            c.wait()
        pltpu.sync_copy(rows_v, out_hbm.at[pl.ds(wid * b_per_w, b_per_w)])

    return k(code_all, idx.reshape(BATCH // 128, 128))


_B_BLK = 512


def _expand_body(code_ref, dict_ref, out_ref):
    lane = lax.broadcasted_iota(jnp.int32, (_B_BLK, DICT_SIZE), 1)
    for l in range(LEVELS):
        c = code_ref[:, l : l + 1]
        oh = (c == lane).astype(jnp.float32)
        out_ref[:, l * FEATURE_DIM : (l + 1) * FEATURE_DIM] = jnp.dot(
            oh, dict_ref[l], preferred_element_type=jnp.float32
        )


def _expand(code_sel, dictionary):
    return pl.pallas_call(
        _expand_body,
        grid=(BATCH // _B_BLK,),
        in_specs=[
            pl.BlockSpec((_B_BLK, _CODE_W), lambda i: (i, 0)),
            pl.BlockSpec((LEVELS, DICT_SIZE, FEATURE_DIM), lambda i: (0, 0, 0)),
        ],
        out_specs=pl.BlockSpec((_B_BLK, LEVELS * FEATURE_DIM), lambda i: (i, 0)),
        out_shape=jax.ShapeDtypeStruct((BATCH, LEVELS * FEATURE_DIM), jnp.float32),
    )(code_sel, dictionary)


def kernel(idx, dictionary, feats):
    code_all = _compute_codes(feats)
    code_sel = _gather_codes(code_all, idx.astype(jnp.int32))
    return _expand(code_sel, dictionary)
